# Initial kernel scaffold; baseline (speedup 1.0000x reference)
import jax
import jax.numpy as jnp
from jax.experimental import pallas as pl


def kernel(x_prev, x_same, x_next, edge_index, gamma, beta, W, b):
    raise NotImplementedError("write your pallas kernel here")


# speedup vs baseline: 15.0757x; 15.0757x over previous
import functools

import jax
import jax.numpy as jnp
from jax import lax
from jax.experimental import pallas as pl
from jax.experimental.pallas import tpu as pltpu
from jax.experimental.pallas import tpu_sc as plsc

N = 10000
E = 320000
D = 128
D2 = 2 * D

NC = 2
NS = 16
NW = NC * NS
EPT = E // NW
C = 80
NCH = EPT // C
NP = 10240
RPT = NP // NS

BN = 400

_mesh = plsc.VectorSubcoreMesh(core_axis_name="c", subcore_axis_name="s")


@functools.partial(
    pl.kernel,
    mesh=_mesh,
    out_type=jax.ShapeDtypeStruct((NC * NP,), jnp.float32),
    scratch_types=[
        pltpu.VMEM((C,), jnp.int32),
        pltpu.VMEM((C,), jnp.float32),
        pltpu.VMEM((RPT,), jnp.float32),
        pltpu.VMEM_SHARED((NP,), jnp.float32),
    ],
)
def _deg_kernel(dst_hbm, out_hbm, idx_v, ones_v, zbuf_v, deg_sh):
    c = lax.axis_index("c")
    s = lax.axis_index("s")
    w = s * NC + c
    for j in range(C // 16):
        ones_v[pl.ds(j * 16, 16)] = jnp.full((16,), 1.0, jnp.float32)
    for j in range(RPT // 16):
        zbuf_v[pl.ds(j * 16, 16)] = jnp.zeros((16,), jnp.float32)
    pltpu.sync_copy(zbuf_v, deg_sh.at[pl.ds(s * RPT, RPT)])
    plsc.subcore_barrier()

    def body(i, carry):
        base = w * EPT + i * C
        pltpu.sync_copy(dst_hbm.at[pl.ds(base, C)], idx_v)
        pltpu.sync_copy(ones_v, deg_sh.at[idx_v], add=True)
        return carry

    lax.fori_loop(0, NCH, body, 0)
    plsc.subcore_barrier()
    pltpu.sync_copy(deg_sh.at[pl.ds(s * RPT, RPT)],
                    out_hbm.at[pl.ds(c * NP + s * RPT, RPT)])


@functools.partial(
    pl.kernel,
    mesh=_mesh,
    out_type=jax.ShapeDtypeStruct((NC * NP, D), jnp.float32),
    scratch_types=[
        pltpu.VMEM((C,), jnp.int32),
        pltpu.VMEM((C,), jnp.int32),
        pltpu.VMEM((C, D), jnp.float32),
        pltpu.VMEM_SHARED((NP, D), jnp.float32),
        pltpu.SemaphoreType.DMA,
    ],
)
def _edge_kernel(g_hbm, src_hbm, dst_hbm, zeros_hbm, out_hbm,
                 src_v, dst_v, rows_v, acc_sh, sem):
    c = lax.axis_index("c")
    s = lax.axis_index("s")
    w = s * NC + c
    pltpu.sync_copy(zeros_hbm.at[pl.ds(s * RPT, RPT)],
                    acc_sh.at[pl.ds(s * RPT, RPT)])
    plsc.subcore_barrier()

    def body(i, carry):
        base = w * EPT + i * C
        pltpu.sync_copy(src_hbm.at[pl.ds(base, C)], src_v)
        pltpu.sync_copy(dst_hbm.at[pl.ds(base, C)], dst_v)
        pltpu.async_copy(g_hbm.at[src_v], rows_v, sem).wait()
        pltpu.sync_copy(rows_v, acc_sh.at[dst_v], add=True)
        return carry

    lax.fori_loop(0, NCH, body, 0)
    plsc.subcore_barrier()
    pltpu.sync_copy(acc_sh.at[pl.ds(s * RPT, RPT)],
                    out_hbm.at[pl.ds(c * NP + s * RPT, RPT)])


def _dense_body(xp_ref, xn_ref, deg2_ref, w_ref, gamma_ref, beta_ref, g_ref):
    gamma = gamma_ref[...]
    beta = beta_ref[...]

    def ln(x):
        mu = jnp.mean(x, axis=-1, keepdims=True)
        xc = x - mu
        var = jnp.mean(xc * xc, axis=-1, keepdims=True)
        return xc * lax.rsqrt(var + 1e-5) * gamma + beta

    xp = ln(xp_ref[...])
    xn = ln(xn_ref[...])
    h = (jnp.dot(xp, w_ref[:D, :], preferred_element_type=jnp.float32)
         + jnp.dot(xn, w_ref[D:, :], preferred_element_type=jnp.float32))
    d2 = deg2_ref[...]
    deg = d2[:, 0] + d2[:, 1] + 1.0
    dinv = lax.rsqrt(deg)
    g_ref[...] = h * dinv[:, None]


_dense_call = pl.pallas_call(
    _dense_body,
    grid=(N // BN,),
    in_specs=[
        pl.BlockSpec((BN, D), lambda i: (i, 0)),
        pl.BlockSpec((BN, D), lambda i: (i, 0)),
        pl.BlockSpec((BN, 2), lambda i: (i, 0)),
        pl.BlockSpec((D2, D), lambda i: (0, 0)),
        pl.BlockSpec((D,), lambda i: (0,)),
        pl.BlockSpec((D,), lambda i: (0,)),
    ],
    out_specs=pl.BlockSpec((BN, D), lambda i: (i, 0)),
    out_shape=jax.ShapeDtypeStruct((N, D), jnp.float32),
)


def _final_body(acc2_ref, g_ref, deg2_ref, b_ref, out_ref):
    acc = acc2_ref[0] + acc2_ref[1]
    d2 = deg2_ref[...]
    deg = d2[:, 0] + d2[:, 1] + 1.0
    dinv = lax.rsqrt(deg)
    out_ref[...] = (acc + g_ref[...]) * dinv[:, None] + b_ref[...]


_final_call = pl.pallas_call(
    _final_body,
    grid=(N // BN,),
    in_specs=[
        pl.BlockSpec((2, BN, D), lambda i: (0, i, 0)),
        pl.BlockSpec((BN, D), lambda i: (i, 0)),
        pl.BlockSpec((BN, 2), lambda i: (i, 0)),
        pl.BlockSpec((D,), lambda i: (0,)),
    ],
    out_specs=pl.BlockSpec((BN, D), lambda i: (i, 0)),
    out_shape=jax.ShapeDtypeStruct((N, D), jnp.float32),
)


@jax.jit
def kernel(x_prev, x_same, x_next, edge_index, gamma, beta, W, b):
    del x_same
    src = edge_index[0]
    dst = edge_index[1]
    degp = _deg_kernel(dst)
    deg2t = degp.reshape(NC, NP)[:, :N].T
    g = _dense_call(x_prev, x_next, deg2t, W, gamma, beta)
    zeros = jnp.zeros((NP, D), jnp.float32)
    accp = _edge_kernel(g, src, dst, zeros)
    acc2 = accp.reshape(NC, NP, D)[:, :N]
    return _final_call(acc2, g, deg2t, b)


# speedup vs baseline: 22.9504x; 1.5224x over previous
import functools

import jax
import jax.numpy as jnp
from jax import lax
from jax.experimental import pallas as pl
from jax.experimental.pallas import tpu as pltpu
from jax.experimental.pallas import tpu_sc as plsc

N = 10000
E = 320000
D = 128
D2 = 2 * D

NC = 2
NS = 16
NW = NC * NS
EPT = E // NW
C = 80
NCH = EPT // C
R = 5
NG = NCH // R
NP = 10240
RPT = NP // NS

BN = 400

_mesh = plsc.VectorSubcoreMesh(core_axis_name="c", subcore_axis_name="s")


@functools.partial(
    pl.kernel,
    mesh=_mesh,
    out_type=jax.ShapeDtypeStruct((NC * NP,), jnp.float32),
    scratch_types=[
        pltpu.VMEM((NCH, C), jnp.int32),
        pltpu.VMEM((C,), jnp.float32),
        pltpu.VMEM((RPT,), jnp.float32),
        pltpu.VMEM_SHARED((NP,), jnp.float32),
        pltpu.SemaphoreType.DMA,
    ],
)
def _deg_kernel(dst_hbm, out_hbm, didx_v, ones_v, zbuf_v, deg_sh, sem):
    c = lax.axis_index("c")
    s = lax.axis_index("s")
    w = s * NC + c
    for j in range(C // 16):
        ones_v[pl.ds(j * 16, 16)] = jnp.full((16,), 1.0, jnp.float32)
    for j in range(RPT // 16):
        zbuf_v[pl.ds(j * 16, 16)] = jnp.zeros((16,), jnp.float32)
    pltpu.sync_copy(dst_hbm.at[w], didx_v)
    pltpu.sync_copy(zbuf_v, deg_sh.at[pl.ds(s * RPT, RPT)])
    plsc.subcore_barrier()

    def body(gi, carry):
        for b in range(NG):
            pltpu.async_copy(ones_v, deg_sh.at[didx_v.at[gi * NG + b]], sem,
                             add=True)
        for b in range(NG):
            pltpu.make_async_copy(ones_v, deg_sh.at[didx_v.at[b]], sem).wait()
        return carry

    lax.fori_loop(0, NCH // NG, body, 0)
    plsc.subcore_barrier()
    pltpu.sync_copy(deg_sh.at[pl.ds(s * RPT, RPT)],
                    out_hbm.at[pl.ds(c * NP + s * RPT, RPT)])


@functools.partial(
    pl.kernel,
    mesh=_mesh,
    out_type=jax.ShapeDtypeStruct((NC * NP, D), jnp.float32),
    scratch_types=[
        pltpu.VMEM((NCH, C), jnp.int32),
        pltpu.VMEM((NCH, C), jnp.int32),
        pltpu.VMEM((C, D), jnp.float32),
        pltpu.VMEM_SHARED((NP, D), jnp.float32),
        pltpu.SemaphoreType.DMA,
    ],
)
def _edge_kernel(g_hbm, src_hbm, dst_hbm, zeros_hbm, out_hbm,
                 sidx_v, didx_v, rows_v, acc_sh, sem):
    c = lax.axis_index("c")
    s = lax.axis_index("s")
    w = s * NC + c
    pltpu.sync_copy(src_hbm.at[w], sidx_v)
    pltpu.sync_copy(dst_hbm.at[w], didx_v)
    pltpu.sync_copy(zeros_hbm.at[pl.ds(s * RPT, RPT)],
                    acc_sh.at[pl.ds(s * RPT, RPT)])
    plsc.subcore_barrier()

    def body(i, carry):
        pltpu.async_copy(g_hbm.at[sidx_v.at[i]], rows_v, sem).wait()
        pltpu.sync_copy(rows_v, acc_sh.at[didx_v.at[i]], add=True)
        return carry

    lax.fori_loop(0, NCH, body, 0)
    plsc.subcore_barrier()
    pltpu.sync_copy(acc_sh.at[pl.ds(s * RPT, RPT)],
                    out_hbm.at[pl.ds(c * NP + s * RPT, RPT)])


def _dense_body(xp_ref, xn_ref, deg2_ref, w_ref, gamma_ref, beta_ref, g_ref):
    gamma = gamma_ref[...]
    beta = beta_ref[...]

    def ln(x):
        mu = jnp.mean(x, axis=-1, keepdims=True)
        xc = x - mu
        var = jnp.mean(xc * xc, axis=-1, keepdims=True)
        return xc * lax.rsqrt(var + 1e-5) * gamma + beta

    xp = ln(xp_ref[...])
    xn = ln(xn_ref[...])
    h = (jnp.dot(xp, w_ref[:D, :], preferred_element_type=jnp.float32)
         + jnp.dot(xn, w_ref[D:, :], preferred_element_type=jnp.float32))
    d2 = deg2_ref[...]
    deg = d2[:, 0] + d2[:, 1] + 1.0
    dinv = lax.rsqrt(deg)
    g_ref[...] = h * dinv[:, None]


_dense_call = pl.pallas_call(
    _dense_body,
    grid=(N // BN,),
    in_specs=[
        pl.BlockSpec((BN, D), lambda i: (i, 0)),
        pl.BlockSpec((BN, D), lambda i: (i, 0)),
        pl.BlockSpec((BN, 2), lambda i: (i, 0)),
        pl.BlockSpec((D2, D), lambda i: (0, 0)),
        pl.BlockSpec((D,), lambda i: (0,)),
        pl.BlockSpec((D,), lambda i: (0,)),
    ],
    out_specs=pl.BlockSpec((BN, D), lambda i: (i, 0)),
    out_shape=jax.ShapeDtypeStruct((N, D), jnp.float32),
)


def _final_body(acc2_ref, g_ref, deg2_ref, b_ref, out_ref):
    acc = acc2_ref[0] + acc2_ref[1]
    d2 = deg2_ref[...]
    deg = d2[:, 0] + d2[:, 1] + 1.0
    dinv = lax.rsqrt(deg)
    out_ref[...] = (acc + g_ref[...]) * dinv[:, None] + b_ref[...]


_final_call = pl.pallas_call(
    _final_body,
    grid=(N // BN,),
    in_specs=[
        pl.BlockSpec((2, BN, D), lambda i: (0, i, 0)),
        pl.BlockSpec((BN, D), lambda i: (i, 0)),
        pl.BlockSpec((BN, 2), lambda i: (i, 0)),
        pl.BlockSpec((D,), lambda i: (0,)),
    ],
    out_specs=pl.BlockSpec((BN, D), lambda i: (i, 0)),
    out_shape=jax.ShapeDtypeStruct((N, D), jnp.float32),
)


@jax.jit
def kernel(x_prev, x_same, x_next, edge_index, gamma, beta, W, b):
    del x_same
    src = edge_index[0].reshape(NW, NCH, C)
    dst = edge_index[1].reshape(NW, NCH, C)
    degp = _deg_kernel(dst)
    deg2t = degp.reshape(NC, NP)[:, :N].T
    g = _dense_call(x_prev, x_next, deg2t, W, gamma, beta)
    zeros = jnp.zeros((NP, D), jnp.float32)
    accp = _edge_kernel(g, src, dst, zeros)
    acc2 = accp.reshape(NC, NP, D)[:, :N]
    return _final_call(acc2, g, deg2t, b)


# speedup vs baseline: 30.9581x; 1.3489x over previous
import functools

import jax
import jax.numpy as jnp
from jax import lax
from jax.experimental import pallas as pl
from jax.experimental.pallas import tpu as pltpu
from jax.experimental.pallas import tpu_sc as plsc

N = 10000
E = 320000
D = 128
D2 = 2 * D

NC = 2
NS = 16
NW = NC * NS
C = 128
NCH = 80
EPT = NCH * C
EP = NW * EPT
NP = 10240
RPT = NP // NS

BN = 400

_mesh = plsc.VectorSubcoreMesh(core_axis_name="c", subcore_axis_name="s")


@functools.partial(
    pl.kernel,
    mesh=_mesh,
    out_type=jax.ShapeDtypeStruct((NC * NP,), jnp.float32),
    scratch_types=[
        pltpu.VMEM((NCH, C), jnp.int32),
        pltpu.VMEM((C,), jnp.float32),
        pltpu.VMEM((RPT,), jnp.float32),
        pltpu.VMEM_SHARED((NP,), jnp.float32),
        pltpu.SemaphoreType.DMA,
    ],
)
def _deg_kernel(dst_hbm, out_hbm, didx_v, ones_v, zbuf_v, deg_sh, sem):
    c = lax.axis_index("c")
    s = lax.axis_index("s")
    w = s * NC + c
    for j in range(C // 16):
        ones_v[pl.ds(j * 16, 16)] = jnp.full((16,), 1.0, jnp.float32)
    for j in range(RPT // 16):
        zbuf_v[pl.ds(j * 16, 16)] = jnp.zeros((16,), jnp.float32)
    pltpu.sync_copy(dst_hbm.at[w], didx_v)
    pltpu.sync_copy(zbuf_v, deg_sh.at[pl.ds(s * RPT, RPT)])
    plsc.subcore_barrier()

    for i in range(NCH):
        pltpu.async_copy(ones_v, deg_sh.at[didx_v.at[i]], sem, add=True)
    for i in range(NCH):
        pltpu.make_async_copy(ones_v, deg_sh.at[didx_v.at[i]], sem).wait()
    plsc.subcore_barrier()
    pltpu.sync_copy(deg_sh.at[pl.ds(s * RPT, RPT)],
                    out_hbm.at[pl.ds(c * NP + s * RPT, RPT)])


@functools.partial(
    pl.kernel,
    mesh=_mesh,
    out_type=jax.ShapeDtypeStruct((NC * NP, D), jnp.float32),
    scratch_types=[
        pltpu.VMEM((NCH, C), jnp.int32),
        pltpu.VMEM((NCH, C), jnp.int32),
        pltpu.VMEM((C, D), jnp.float32),
        pltpu.VMEM_SHARED((NP, D), jnp.float32),
        pltpu.SemaphoreType.DMA,
    ],
)
def _edge_kernel(g_hbm, src_hbm, dst_hbm, zeros_hbm, out_hbm,
                 sidx_v, didx_v, rows_v, acc_sh, sem):
    c = lax.axis_index("c")
    s = lax.axis_index("s")
    w = s * NC + c
    pltpu.sync_copy(src_hbm.at[w], sidx_v)
    pltpu.sync_copy(dst_hbm.at[w], didx_v)
    pltpu.sync_copy(zeros_hbm.at[pl.ds(s * RPT, RPT)],
                    acc_sh.at[pl.ds(s * RPT, RPT)])
    plsc.subcore_barrier()

    def body(i, carry):
        pltpu.async_copy(g_hbm.at[sidx_v.at[i]], rows_v, sem).wait()
        pltpu.sync_copy(rows_v, acc_sh.at[didx_v.at[i]], add=True)
        return carry

    lax.fori_loop(0, NCH, body, 0)
    plsc.subcore_barrier()
    pltpu.sync_copy(acc_sh.at[pl.ds(s * RPT, RPT)],
                    out_hbm.at[pl.ds(c * NP + s * RPT, RPT)])


def _dense_body(xp_ref, xn_ref, deg2_ref, w_ref, gamma_ref, beta_ref, g_ref):
    gamma = gamma_ref[...]
    beta = beta_ref[...]

    def ln(x):
        mu = jnp.mean(x, axis=-1, keepdims=True)
        xc = x - mu
        var = jnp.mean(xc * xc, axis=-1, keepdims=True)
        return xc * lax.rsqrt(var + 1e-5) * gamma + beta

    xp = ln(xp_ref[...])
    xn = ln(xn_ref[...])
    h = (jnp.dot(xp, w_ref[:D, :], preferred_element_type=jnp.float32)
         + jnp.dot(xn, w_ref[D:, :], preferred_element_type=jnp.float32))
    d2 = deg2_ref[...]
    deg = d2[:, 0] + d2[:, 1] + 1.0
    dinv = lax.rsqrt(deg)
    g_ref[...] = h * dinv[:, None]


_dense_call = pl.pallas_call(
    _dense_body,
    grid=(N // BN,),
    in_specs=[
        pl.BlockSpec((BN, D), lambda i: (i, 0)),
        pl.BlockSpec((BN, D), lambda i: (i, 0)),
        pl.BlockSpec((BN, 2), lambda i: (i, 0)),
        pl.BlockSpec((D2, D), lambda i: (0, 0)),
        pl.BlockSpec((D,), lambda i: (0,)),
        pl.BlockSpec((D,), lambda i: (0,)),
    ],
    out_specs=pl.BlockSpec((BN, D), lambda i: (i, 0)),
    out_shape=jax.ShapeDtypeStruct((N, D), jnp.float32),
)


def _final_body(acc2_ref, g_ref, deg2_ref, b_ref, out_ref):
    acc = acc2_ref[0] + acc2_ref[1]
    d2 = deg2_ref[...]
    deg = d2[:, 0] + d2[:, 1] + 1.0
    dinv = lax.rsqrt(deg)
    out_ref[...] = (acc + g_ref[...]) * dinv[:, None] + b_ref[...]


_final_call = pl.pallas_call(
    _final_body,
    grid=(N // BN,),
    in_specs=[
        pl.BlockSpec((2, BN, D), lambda i: (0, i, 0)),
        pl.BlockSpec((BN, D), lambda i: (i, 0)),
        pl.BlockSpec((BN, 2), lambda i: (i, 0)),
        pl.BlockSpec((D,), lambda i: (0,)),
    ],
    out_specs=pl.BlockSpec((BN, D), lambda i: (i, 0)),
    out_shape=jax.ShapeDtypeStruct((N, D), jnp.float32),
)


@jax.jit
def kernel(x_prev, x_same, x_next, edge_index, gamma, beta, W, b):
    del x_same
    pad = EP - E
    pad_src = (jnp.arange(pad, dtype=jnp.int32) * 37) % N
    pad_dst = N + (jnp.arange(pad, dtype=jnp.int32) % (NP - N))
    src = jnp.concatenate([edge_index[0], pad_src]).reshape(NW, NCH, C)
    dst = jnp.concatenate([edge_index[1], pad_dst]).reshape(NW, NCH, C)
    degp = _deg_kernel(dst)
    deg2t = degp.reshape(NC, NP)[:, :N].T
    g = _dense_call(x_prev, x_next, deg2t, W, gamma, beta)
    zeros = jnp.zeros((NP, D), jnp.float32)
    accp = _edge_kernel(g, src, dst, zeros)
    acc2 = accp.reshape(NC, NP, D)[:, :N]
    return _final_call(acc2, g, deg2t, b)
